# Initial kernel scaffold; baseline (speedup 1.0000x reference)
import jax
import jax.numpy as jnp
from jax.experimental import pallas as pl


def kernel(inputs, input_axon_embeddings, scale, bias, keys_idx):
    raise NotImplementedError("write your pallas kernel here")


# speedup vs baseline: 3.3705x; 3.3705x over previous
import functools

import jax
import jax.numpy as jnp
from jax import lax
from jax.experimental import pallas as pl
from jax.experimental.pallas import tpu as pltpu
from jax.experimental.pallas import tpu_sc as plsc

N_NEURONS = 1000
EMBED_DIM = 128
BATCH = 1024
N_SELECTED = 16384

NC, NS, L = 2, 16, 16
NW = NC * NS
B_PER_W = N_SELECTED // NW
AUX_W = 128
CHUNK = 256


def _sc_gather_body(aug_hbm, keys_hbm,
                    emb_hbm, aux_g_hbm,
                    idx_v, rows_v, sem):
    wid = lax.axis_index("s") * NC + lax.axis_index("c")
    base = wid * B_PER_W
    pltpu.sync_copy(keys_hbm.at[pl.ds(base, B_PER_W)], idx_v)
    for c in range(B_PER_W // CHUNK):
        pltpu.async_copy(
            aug_hbm.at[idx_v.at[pl.ds(c * CHUNK, CHUNK)]], rows_v, sem
        ).wait()
        pltpu.sync_copy(rows_v.at[:, pl.ds(0, EMBED_DIM)],
                        emb_hbm.at[pl.ds(base + c * CHUNK, CHUNK)])
        pltpu.sync_copy(rows_v.at[:, pl.ds(EMBED_DIM, AUX_W)],
                        aux_g_hbm.at[pl.ds(base + c * CHUNK, CHUNK)])


@functools.cache
def _sc_gather():
    return pl.kernel(
        _sc_gather_body,
        out_type=(
            jax.ShapeDtypeStruct((N_SELECTED, EMBED_DIM), jnp.float32),
            jax.ShapeDtypeStruct((N_SELECTED, AUX_W), jnp.float32),
        ),
        mesh=plsc.VectorSubcoreMesh(core_axis_name="c", subcore_axis_name="s",
                                    num_cores=NC, num_subcores=NS),
        scratch_types=[
            pltpu.VMEM((B_PER_W,), jnp.int32),
            pltpu.VMEM((CHUNK, EMBED_DIM + AUX_W), jnp.float32),
            pltpu.SemaphoreType.DMA,
        ],
    )


def _tc_affine_body(x_ref, s_ref, b_ref, o_ref):
    o_ref[...] = b_ref[...] + s_ref[...] * x_ref[...]


ROW_BLK = 256
COL_BLK = 4096

_tc_affine = pl.pallas_call(
    _tc_affine_body,
    grid=(BATCH // ROW_BLK, N_SELECTED // COL_BLK),
    in_specs=[
        pl.BlockSpec((ROW_BLK, COL_BLK), lambda i, j: (i, j)),
        pl.BlockSpec((1, COL_BLK), lambda i, j: (0, j)),
        pl.BlockSpec((1, COL_BLK), lambda i, j: (0, j)),
    ],
    out_specs=pl.BlockSpec((ROW_BLK, COL_BLK), lambda i, j: (i, j)),
    out_shape=jax.ShapeDtypeStruct((BATCH, N_SELECTED), jnp.float32),
)


def kernel(inputs, input_axon_embeddings, scale, bias, keys_idx):
    aug = jnp.concatenate(
        [input_axon_embeddings, scale[:, None], bias[:, None],
         jnp.zeros((N_NEURONS, AUX_W - 2), jnp.float32)], axis=1)
    out_emb, aux_g = _sc_gather()(aug, keys_idx.astype(jnp.int32))
    scale_g = aux_g[:, 0].reshape(1, N_SELECTED)
    bias_g = aux_g[:, 1].reshape(1, N_SELECTED)
    out_inputs = _tc_affine(inputs, scale_g, bias_g)
    return (out_inputs, out_emb)


# speedup vs baseline: 3.6242x; 1.0753x over previous
import functools

import jax
import jax.numpy as jnp
from jax import lax
from jax.experimental import pallas as pl
from jax.experimental.pallas import tpu as pltpu
from jax.experimental.pallas import tpu_sc as plsc

N_NEURONS = 1000
EMBED_DIM = 128
BATCH = 1024
N_SELECTED = 16384

NC, NS, L = 2, 16, 16
NW = NC * NS
B_PER_W = N_SELECTED // NW
AUX_W = 128


def _worker_base():
    wid = lax.axis_index("s") * NC + lax.axis_index("c")
    return wid * B_PER_W


def _sc_aux_body(aux_hbm, keys_hbm, aux_g_hbm, idx_v, rows_v, sem):
    base = _worker_base()
    pltpu.sync_copy(keys_hbm.at[pl.ds(base, B_PER_W)], idx_v)
    pltpu.async_copy(aux_hbm.at[idx_v], rows_v, sem).wait()
    pltpu.sync_copy(rows_v, aux_g_hbm.at[pl.ds(base, B_PER_W)])


def _sc_emb_body(table_hbm, keys_hbm, emb_hbm, idx_v, rows_v, sem):
    base = _worker_base()
    pltpu.sync_copy(keys_hbm.at[pl.ds(base, B_PER_W)], idx_v)
    pltpu.async_copy(table_hbm.at[idx_v], rows_v, sem).wait()
    pltpu.sync_copy(rows_v, emb_hbm.at[pl.ds(base, B_PER_W)])


def _sc_mesh():
    return plsc.VectorSubcoreMesh(core_axis_name="c", subcore_axis_name="s",
                                  num_cores=NC, num_subcores=NS)


@functools.cache
def _sc_aux():
    return pl.kernel(
        _sc_aux_body,
        out_type=jax.ShapeDtypeStruct((N_SELECTED, AUX_W), jnp.float32),
        mesh=_sc_mesh(),
        scratch_types=[
            pltpu.VMEM((B_PER_W,), jnp.int32),
            pltpu.VMEM((B_PER_W, AUX_W), jnp.float32),
            pltpu.SemaphoreType.DMA,
        ],
    )


@functools.cache
def _sc_emb():
    return pl.kernel(
        _sc_emb_body,
        out_type=jax.ShapeDtypeStruct((N_SELECTED, EMBED_DIM), jnp.float32),
        mesh=_sc_mesh(),
        scratch_types=[
            pltpu.VMEM((B_PER_W,), jnp.int32),
            pltpu.VMEM((B_PER_W, EMBED_DIM), jnp.float32),
            pltpu.SemaphoreType.DMA,
        ],
    )


def _tc_affine_body(x_ref, s_ref, b_ref, o_ref):
    o_ref[...] = b_ref[...] + s_ref[...] * x_ref[...]


ROW_BLK = 512
COL_BLK = 4096

_tc_affine = pl.pallas_call(
    _tc_affine_body,
    grid=(BATCH // ROW_BLK, N_SELECTED // COL_BLK),
    in_specs=[
        pl.BlockSpec((ROW_BLK, COL_BLK), lambda i, j: (i, j)),
        pl.BlockSpec((1, COL_BLK), lambda i, j: (0, j)),
        pl.BlockSpec((1, COL_BLK), lambda i, j: (0, j)),
    ],
    out_specs=pl.BlockSpec((ROW_BLK, COL_BLK), lambda i, j: (i, j)),
    out_shape=jax.ShapeDtypeStruct((BATCH, N_SELECTED), jnp.float32),
)


def kernel(inputs, input_axon_embeddings, scale, bias, keys_idx):
    aux = jnp.concatenate(
        [scale[:, None], bias[:, None],
         jnp.zeros((N_NEURONS, AUX_W - 2), jnp.float32)], axis=1)
    keys32 = keys_idx.astype(jnp.int32)
    aux_g = _sc_aux()(aux, keys32)
    out_emb = _sc_emb()(input_axon_embeddings, keys32)
    scale_g = aux_g[:, 0].reshape(1, N_SELECTED)
    bias_g = aux_g[:, 1].reshape(1, N_SELECTED)
    out_inputs = _tc_affine(inputs, scale_g, bias_g)
    return (out_inputs, out_emb)
